# baseline (speedup 1.0000x reference)
import functools

import jax
import jax.numpy as jnp
from jax import lax
from jax.experimental import pallas as pl
from jax.experimental.pallas import tpu as pltpu
from jax.experimental.pallas import tpu_sc as plsc

_NC = 2
_NS = 16
_NW = _NC * _NS

_SEQ = 200
_D = 64
_LANES = 16

_GATHER_N = 100
_CHUNK = 400
_GATHERS_PER_CHUNK = _CHUNK // _GATHER_N


def _sc_embed(table, idx, pos, *, n_per_w, n_chunks):
    mesh = plsc.VectorSubcoreMesh(core_axis_name="c", subcore_axis_name="s")

    @functools.partial(
        pl.kernel,
        mesh=mesh,
        out_type=jax.ShapeDtypeStruct((_NW * n_per_w, _D), jnp.float32),
        compiler_params=pltpu.CompilerParams(use_tc_tiling_on_sc=False),
        scratch_types=[
            pltpu.VMEM((n_per_w // _GATHER_N, _GATHER_N), jnp.int32),
            pltpu.VMEM((_SEQ, _D), jnp.float32),
            pltpu.VMEM((_CHUNK, _D), jnp.float32),
            pltpu.VMEM((_CHUNK, _D), jnp.float32),
            pltpu.SemaphoreType.DMA,
            pltpu.SemaphoreType.DMA,
            pltpu.SemaphoreType.DMA,
            pltpu.SemaphoreType.DMA,
        ],
    )
    def k(table_hbm, idx_hbm, pos_hbm, out_hbm,
          idx_v, pos_v, buf0, buf1, sg0, sg1, so0, so1):
        wid = lax.axis_index("s") * _NC + lax.axis_index("c")
        base = wid * n_per_w

        pltpu.sync_copy(idx_hbm.at[wid], idx_v)
        pltpu.sync_copy(pos_hbm, pos_v)

        bufs = (buf0, buf1)
        gsems = (sg0, sg1)
        osems = (so0, so1)

        def start_gathers(c):
            buf = bufs[c % 2]
            sem = gsems[c % 2]
            descs = []
            for j in range(_GATHERS_PER_CHUNK):
                row = c * _GATHERS_PER_CHUNK + j
                descs.append(pltpu.async_copy(
                    table_hbm.at[idx_v.at[row]],
                    buf.at[pl.ds(j * _GATHER_N, _GATHER_N)],
                    sem))
            return descs

        def compute(c):
            buf = bufs[c % 2]

            def body(r, _):
                for h in range(_CHUNK // _SEQ):
                    row = buf.at[h * _SEQ + r]
                    prow = pos_v.at[r]
                    for t in range(_D // _LANES):
                        sl = pl.ds(t * _LANES, _LANES)
                        row[sl] = row[sl] * 8.0 + prow[sl]
                return 0

            lax.fori_loop(0, _SEQ, body, 0)

        def start_out(c):
            buf = bufs[c % 2]
            sem = osems[c % 2]
            return pltpu.async_copy(
                buf, out_hbm.at[pl.ds(base + c * _CHUNK, _CHUNK)], sem)

        out_descs = [None, None]
        gather_descs = start_gathers(0)
        for c in range(n_chunks):
            if c + 1 < n_chunks:
                if out_descs[(c + 1) % 2] is not None:
                    out_descs[(c + 1) % 2].wait()
                    out_descs[(c + 1) % 2] = None
                next_descs = start_gathers(c + 1)
            else:
                next_descs = None
            for d in gather_descs:
                d.wait()
            gather_descs = next_descs
            compute(c)
            out_descs[c % 2] = start_out(c)
        for d in out_descs:
            if d is not None:
                d.wait()

    return k(table, idx, pos)


def kernel(inp, table, pos_encoding):
    batch, seq = inp.shape
    d = table.shape[1]
    total = batch * seq
    n_per_w = total // _NW
    n_chunks = n_per_w // _CHUNK
    idx = inp.reshape(_NW, n_per_w // _GATHER_N, _GATHER_N)
    pos = pos_encoding[0, :seq, :]
    out = _sc_embed(table, idx, pos, n_per_w=n_per_w, n_chunks=n_chunks)
    return out.reshape(batch, seq, d)
